# Initial kernel scaffold; baseline (speedup 1.0000x reference)
import jax
import jax.numpy as jnp
from jax.experimental import pallas as pl


def kernel(x):
    raise NotImplementedError("write your pallas kernel here")


# speedup vs baseline: 1.1355x; 1.1355x over previous
import math

import numpy as np

import jax
import jax.numpy as jnp
from jax.experimental import pallas as pl
from jax.experimental.pallas import tpu as pltpu


def _interp_arrays(out_size, in_size):
    if out_size == 1:
        src = np.zeros((1,), np.float32)
    else:
        src = np.arange(out_size, dtype=np.float32) * np.float32(
            (in_size - 1) / (out_size - 1)
        )
    i0 = np.clip(np.floor(src).astype(np.int32), 0, in_size - 1)
    i1 = np.minimum(i0 + 1, in_size - 1)
    w1 = src - i0.astype(np.float32)
    w0 = np.float32(1.0) - w1
    return i0, i1, w0, w1


def _pair_coeffs(out_size, in_size, scale):
    i0, i1, w0, w1 = _interp_arrays(out_size, in_size)
    ho = np.arange(out_size)
    assert np.all((i0 == 2 * ho) | (i0 == 2 * ho + 1))
    assert np.all((i1 == 2 * ho) | (i1 == 2 * ho + 1))
    a0 = np.where(i0 == 2 * ho, w0, 0.0) + np.where(i1 == 2 * ho, w1, 0.0)
    a1 = np.where(i0 == 2 * ho + 1, w0, 0.0) + np.where(i1 == 2 * ho + 1, w1, 0.0)
    return (np.float32(scale) * a0.astype(np.float32),
            np.float32(scale) * a1.astype(np.float32))


def _interp_matrix_t(out_size, in_size):
    i0, i1, w0, w1 = _interp_arrays(out_size, in_size)
    m = np.zeros((out_size, in_size), np.float32)
    m[np.arange(out_size), i0] += w0
    m[np.arange(out_size), i1] += w1
    return np.ascontiguousarray(m.T)


def _resize_kernel(x_ref, wwt_ref, a0_ref, a1_ref, o_ref):
    x = x_ref[...]
    w = x.shape[2] // 2
    t = a0_ref[...] * x[:, :, :w] + a1_ref[...] * x[:, :, w:]
    tb, ho, _ = t.shape
    wo = wwt_ref.shape[1]
    y = jnp.dot(t.reshape(tb * ho, w), wwt_ref[...],
                preferred_element_type=jnp.float32)
    o_ref[...] = y.reshape(tb, ho, wo)


def kernel(x):
    vel_resize = 2.0
    factor = 1.0 / vel_resize
    N, C, H, W = x.shape
    H_out = int(math.floor(H * factor))
    W_out = int(math.floor(W * factor))
    assert H == 2 * H_out and W == 2 * W_out
    B = N * C

    a0, a1 = _pair_coeffs(H_out, H, factor)
    wwt = jnp.asarray(_interp_matrix_t(W_out, W))
    a0 = jnp.asarray(a0).reshape(1, H_out, 1)
    a1 = jnp.asarray(a1).reshape(1, H_out, 1)

    TB = 8
    assert B % TB == 0
    grid_b = B // TB

    out3 = pl.pallas_call(
        _resize_kernel,
        out_shape=jax.ShapeDtypeStruct((B, H_out, W_out), x.dtype),
        grid=(grid_b,),
        in_specs=[
            pl.BlockSpec((TB, H_out, 2 * W), lambda b: (b, 0, 0)),
            pl.BlockSpec((W, W_out), lambda b: (0, 0)),
            pl.BlockSpec((1, H_out, 1), lambda b: (0, 0, 0)),
            pl.BlockSpec((1, H_out, 1), lambda b: (0, 0, 0)),
        ],
        out_specs=pl.BlockSpec((TB, H_out, W_out), lambda b: (b, 0, 0)),
        compiler_params=pltpu.CompilerParams(
            dimension_semantics=("parallel",),
            vmem_limit_bytes=int(64 * 1024 * 1024 * 0.85),
        ),
    )(x.reshape(B, H_out, 2 * W), wwt, a0, a1)
    return out3.reshape(N, C, H_out, W_out)


# speedup vs baseline: 1.2344x; 1.0871x over previous
import math

import numpy as np

import jax
import jax.numpy as jnp
from jax.experimental import pallas as pl
from jax.experimental.pallas import tpu as pltpu


def _interp_arrays(out_size, in_size):
    if out_size == 1:
        src = np.zeros((1,), np.float32)
    else:
        src = np.arange(out_size, dtype=np.float32) * np.float32(
            (in_size - 1) / (out_size - 1)
        )
    i0 = np.clip(np.floor(src).astype(np.int32), 0, in_size - 1)
    i1 = np.minimum(i0 + 1, in_size - 1)
    w1 = src - i0.astype(np.float32)
    w0 = np.float32(1.0) - w1
    return i0, i1, w0, w1


def _pair_coeffs(out_size, in_size, scale):
    i0, i1, w0, w1 = _interp_arrays(out_size, in_size)
    ho = np.arange(out_size)
    assert np.all((i0 == 2 * ho) | (i0 == 2 * ho + 1))
    assert np.all((i1 == 2 * ho) | (i1 == 2 * ho + 1))
    a0 = np.where(i0 == 2 * ho, w0, 0.0) + np.where(i1 == 2 * ho, w1, 0.0)
    a1 = np.where(i0 == 2 * ho + 1, w0, 0.0) + np.where(i1 == 2 * ho + 1, w1, 0.0)
    return (np.float32(scale) * a0.astype(np.float32),
            np.float32(scale) * a1.astype(np.float32))


def _interp_matrix_t(out_size, in_size):
    i0, i1, w0, w1 = _interp_arrays(out_size, in_size)
    m = np.zeros((out_size, in_size), np.float32)
    m[np.arange(out_size), i0] += w0
    m[np.arange(out_size), i1] += w1
    return np.ascontiguousarray(m.T)


def _resize_kernel(xa_ref, xb_ref, wwt_ref, a0_ref, a1_ref, o_ref):
    a0 = a0_ref[...]
    a1 = a1_ref[...]
    wwt = wwt_ref[...]
    wo = wwt.shape[1]
    for half, x_ref in ((0, xa_ref), (1, xb_ref)):
        x = x_ref[...]
        tb, ho, w2 = x.shape
        w = w2 // 2
        t = a0 * x[:, :, :w] + a1 * x[:, :, w:]
        y = jnp.dot(t.reshape(tb * ho, w), wwt,
                    preferred_element_type=jnp.float32)
        o_ref[half * tb:(half + 1) * tb] = y.reshape(tb, ho, wo)


def kernel(x):
    vel_resize = 2.0
    factor = 1.0 / vel_resize
    N, C, H, W = x.shape
    H_out = int(math.floor(H * factor))
    W_out = int(math.floor(W * factor))
    assert H == 2 * H_out and W == 2 * W_out
    B = N * C

    a0, a1 = _pair_coeffs(H_out, H, factor)
    wwt = jnp.asarray(_interp_matrix_t(W_out, W))
    a0 = jnp.asarray(a0).reshape(1, H_out, 1)
    a1 = jnp.asarray(a1).reshape(1, H_out, 1)

    TB = 8
    while TB > 1 and B % (2 * TB):
        TB //= 2
    assert B % (2 * TB) == 0
    grid_b = B // (2 * TB)

    out3 = pl.pallas_call(
        _resize_kernel,
        out_shape=jax.ShapeDtypeStruct((B, H_out, W_out), x.dtype),
        grid=(grid_b,),
        in_specs=[
            pl.BlockSpec((TB, H_out, 2 * W), lambda b: (2 * b, 0, 0)),
            pl.BlockSpec((TB, H_out, 2 * W), lambda b: (2 * b + 1, 0, 0)),
            pl.BlockSpec((W, W_out), lambda b: (0, 0)),
            pl.BlockSpec((1, H_out, 1), lambda b: (0, 0, 0)),
            pl.BlockSpec((1, H_out, 1), lambda b: (0, 0, 0)),
        ],
        out_specs=pl.BlockSpec((2 * TB, H_out, W_out), lambda b: (b, 0, 0)),
        compiler_params=pltpu.CompilerParams(
            dimension_semantics=("parallel",),
            vmem_limit_bytes=int(64 * 1024 * 1024 * 0.85),
        ),
    )(x.reshape(B, H_out, 2 * W), x.reshape(B, H_out, 2 * W), wwt, a0, a1)
    return out3.reshape(N, C, H_out, W_out)


# speedup vs baseline: 3.2922x; 2.6671x over previous
import math

import numpy as np

import jax
import jax.numpy as jnp
from jax.experimental import pallas as pl
from jax.experimental.pallas import tpu as pltpu


def _interp_arrays(out_size, in_size):
    if out_size == 1:
        src = np.zeros((1,), np.float32)
    else:
        src = np.arange(out_size, dtype=np.float32) * np.float32(
            (in_size - 1) / (out_size - 1)
        )
    i0 = np.clip(np.floor(src).astype(np.int32), 0, in_size - 1)
    i1 = np.minimum(i0 + 1, in_size - 1)
    w1 = src - i0.astype(np.float32)
    w0 = np.float32(1.0) - w1
    return i0, i1, w0, w1


def _interp_matrix(out_size, in_size):
    i0, i1, w0, w1 = _interp_arrays(out_size, in_size)
    m = np.zeros((out_size, in_size), np.float32)
    m[np.arange(out_size), i0] += w0
    m[np.arange(out_size), i1] += w1
    return m


def _resize_kernel(xa_ref, xb_ref, wwt_ref, wh_ref, o_ref):
    wwt = wwt_ref[...]
    wh = wh_ref[...]
    wo = wwt.shape[1]
    for half, x_ref in ((0, xa_ref), (1, xb_ref)):
        x = x_ref[...]
        tb, h, w = x.shape
        u = jnp.dot(x.reshape(tb * h, w), wwt,
                    preferred_element_type=jnp.float32).reshape(tb, h, wo)
        for b in range(tb):
            o_ref[half * tb + b] = jnp.dot(wh, u[b],
                                           preferred_element_type=jnp.float32)


def kernel(x):
    vel_resize = 2.0
    factor = 1.0 / vel_resize
    N, C, H, W = x.shape
    H_out = int(math.floor(H * factor))
    W_out = int(math.floor(W * factor))
    B = N * C

    wwt = jnp.asarray(np.ascontiguousarray(_interp_matrix(W_out, W).T))
    wh = jnp.asarray(np.float32(factor) * _interp_matrix(H_out, H))

    TB = 8
    while TB > 1 and B % (2 * TB):
        TB //= 2
    assert B % (2 * TB) == 0
    grid_b = B // (2 * TB)

    xf = x.reshape(B, H, W)
    out3 = pl.pallas_call(
        _resize_kernel,
        out_shape=jax.ShapeDtypeStruct((B, H_out, W_out), x.dtype),
        grid=(grid_b,),
        in_specs=[
            pl.BlockSpec((TB, H, W), lambda b: (2 * b, 0, 0)),
            pl.BlockSpec((TB, H, W), lambda b: (2 * b + 1, 0, 0)),
            pl.BlockSpec((W, W_out), lambda b: (0, 0)),
            pl.BlockSpec((H_out, H), lambda b: (0, 0)),
        ],
        out_specs=pl.BlockSpec((2 * TB, H_out, W_out), lambda b: (b, 0, 0)),
        compiler_params=pltpu.CompilerParams(
            dimension_semantics=("parallel",),
            vmem_limit_bytes=int(64 * 1024 * 1024 * 0.85),
        ),
    )(xf, xf, wwt, wh)
    return out3.reshape(N, C, H_out, W_out)
